# baseline (speedup 1.0000x reference)
import jax
import jax.numpy as jnp
from jax import lax
from jax.experimental import pallas as pl
from jax.experimental.pallas import tpu as pltpu

R, N = 64, 32768
LANES = 128
BNC = 4096
NC = N // BNC
CHUNKS = BNC // LANES
U = 2
CG = 16
TG = CHUNKS // CG

_INT_MAX = 2**31 - 1


P = 4
RP = R // P


def _tc_body(x_hbm, o_ref, buf0, buf1, sems):
    bufs = (buf0, buf1)

    def subcopies(c):
        b = c % 2
        return [pltpu.make_async_copy(
            x_hbm.at[pl.ds(p * RP, RP), pl.ds(c * BNC, BNC)],
            bufs[b].at[pl.ds(p * RP, RP), :],
            sems.at[b, p]) for p in range(P)]

    def start(c):
        for cp in subcopies(c):
            cp.start()

    def wait(c):
        for cp in subcopies(c):
            cp.wait()

    start(0)
    rvs = [jnp.full((R, LANES), -jnp.inf, jnp.float32) for _ in range(U)]
    ris = [jnp.zeros((R, LANES), jnp.int32) for _ in range(U)]

    for c in range(NC):
        if c + 1 < NC:
            start(c + 1)
        wait(c)
        buf = bufs[c % 2]

        def group(t, carry, c=c, buf=buf):
            rvs, ris = carry
            rvs, ris = list(rvs), list(ris)
            base = t * CG
            for jj in range(CG):
                k = jj % U
                chunk = buf[:, pl.ds((base + jj) * LANES, LANES)]
                m = chunk > rvs[k]
                rvs[k] = jnp.where(m, chunk, rvs[k])
                ris[k] = jnp.where(m, c * CHUNKS + base + jj, ris[k])
            return tuple(rvs), tuple(ris)

        rvs, ris = lax.fori_loop(0, TG, group, (tuple(rvs), tuple(ris)))
        rvs, ris = list(rvs), list(ris)

    lane = lax.broadcasted_iota(jnp.int32, (R, LANES), 1)
    pairs = [(rvs[k], ris[k] * LANES + lane) for k in range(U)]
    while len(pairs) > 1:
        nxt = []
        for a in range(0, len(pairs), 2):
            (va, ia), (vb, ib) = pairs[a], pairs[a + 1]
            take_b = (vb > va) | ((vb == va) & (ib < ia))
            nxt.append((jnp.where(take_b, vb, va),
                        jnp.where(take_b, ib, ia)))
        pairs = nxt
    rv, ri = pairs[0]
    mx = jnp.max(rv, axis=1, keepdims=True)
    cand = jnp.where(rv == mx, ri, _INT_MAX)
    o_ref[...] = jnp.min(cand, axis=1)[None, :]


@jax.jit
def _argmax_rows(x):
    out = pl.pallas_call(
        _tc_body,
        in_specs=[pl.BlockSpec(memory_space=pltpu.HBM)],
        out_specs=pl.BlockSpec((1, R), lambda: (0, 0)),
        out_shape=jax.ShapeDtypeStruct((1, R), jnp.int32),
        scratch_shapes=[
            pltpu.VMEM((R, BNC), jnp.float32),
            pltpu.VMEM((R, BNC), jnp.float32),
            pltpu.SemaphoreType.DMA((2, P)),
        ],
    )(x)
    return out.reshape(R)


def kernel(x):
    return _argmax_rows(x)


# speedup vs baseline: 1.4734x; 1.4734x over previous
import jax
import jax.numpy as jnp
from jax import lax
from jax.experimental import pallas as pl
from jax.experimental.pallas import tpu as pltpu

R, N = 64, 32768
LANES = 128
BN = 16384
GRID = N // BN
CHUNKS = BN // LANES
U = 2
CG = 32
TG = CHUNKS // CG

_INT_MAX = 2**31 - 1


def _tc_body(x_ref, o_ref, rv_ref, ri_ref):
    i = pl.program_id(0)

    def first():
        return (tuple(jnp.full((R, LANES), -jnp.inf, jnp.float32)
                      for _ in range(U)),
                tuple(jnp.zeros((R, LANES), jnp.int32) for _ in range(U)))

    def later():
        return (tuple(rv_ref[k] for k in range(U)),
                tuple(ri_ref[k] for k in range(U)))

    rvs, ris = lax.cond(i == 0, first, later)
    rvs, ris = list(rvs), list(ris)

    def group(t, carry):
        rvs, ris = carry
        rvs, ris = list(rvs), list(ris)
        base = t * CG
        for jj in range(CG):
            k = jj % U
            chunk = x_ref[:, pl.ds((base + jj) * LANES, LANES)]
            m = chunk > rvs[k]
            rvs[k] = jnp.where(m, chunk, rvs[k])
            ris[k] = jnp.where(m, i * CHUNKS + base + jj, ris[k])
        return tuple(rvs), tuple(ris)

    rvs, ris = lax.fori_loop(0, TG, group, (tuple(rvs), tuple(ris)))

    for k in range(U):
        rv_ref[k] = rvs[k]
        ri_ref[k] = ris[k]

    @pl.when(i == GRID - 1)
    def _finish():
        lane = lax.broadcasted_iota(jnp.int32, (R, LANES), 1)
        pairs = [(rvs[k], ris[k] * LANES + lane) for k in range(U)]
        while len(pairs) > 1:
            nxt = []
            for a in range(0, len(pairs), 2):
                (va, ia), (vb, ib) = pairs[a], pairs[a + 1]
                take_b = (vb > va) | ((vb == va) & (ib < ia))
                nxt.append((jnp.where(take_b, vb, va),
                            jnp.where(take_b, ib, ia)))
            pairs = nxt
        rv, ri = pairs[0]
        mx = jnp.max(rv, axis=1, keepdims=True)
        cand = jnp.where(rv == mx, ri, _INT_MAX)
        o_ref[...] = jnp.min(cand, axis=1)[None, :]


@jax.jit
def _argmax_rows(x):
    out = pl.pallas_call(
        _tc_body,
        grid=(GRID,),
        in_specs=[pl.BlockSpec((R, BN), lambda i: (0, i))],
        out_specs=pl.BlockSpec((1, R), lambda i: (0, 0)),
        out_shape=jax.ShapeDtypeStruct((1, R), jnp.int32),
        scratch_shapes=[
            pltpu.VMEM((U, R, LANES), jnp.float32),
            pltpu.VMEM((U, R, LANES), jnp.int32),
        ],
    )(x)
    return out.reshape(R)


def kernel(x):
    return _argmax_rows(x)
